# Initial kernel scaffold; baseline (speedup 1.0000x reference)
import jax
import jax.numpy as jnp
from jax.experimental import pallas as pl


def kernel(x, edge_index, Wc1, al1, ar1, bc1, Wc2, al2, ar2, bc2, W1, b1, W2, b2, W3, b3):
    raise NotImplementedError("write your pallas kernel here")


# speedup vs baseline: 1.0000x; 1.0000x over previous
import jax
import jax.numpy as jnp
from jax.experimental import pallas as pl

N = 10000
E = 320000
IN_DIM = 128
HID = 32
HEADS = 8
NCLS = 10


def _gat_conv(x, src, dst, W, al, ar, b):
    n = x.shape[0]
    feat = (x @ W).reshape(n, HEADS, HID)
    el = (feat * al[None, :, :]).sum(-1)
    er = (feat * ar[None, :, :]).sum(-1)
    e = jax.nn.leaky_relu(el[src] + er[dst], negative_slope=0.2)
    emax = jax.ops.segment_max(e, dst, num_segments=n)
    ex = jnp.exp(e - emax[dst])
    denom = jax.ops.segment_sum(ex, dst, num_segments=n)
    d = denom[dst]
    alpha = ex / jnp.where(d > 0, d, 1.0)
    msg = feat[src] * alpha[..., None]
    out = jax.ops.segment_sum(msg, dst, num_segments=n)
    return out + b.reshape(1, HEADS, HID)


def _mlp_kernel(hg_ref, W1_ref, b1_ref, W2_ref, b2_ref, W3_ref, b3_ref, o_ref):
    hg = hg_ref[...]
    o = jax.nn.relu(hg @ W1_ref[...] + b1_ref[...][None, :])
    o = jax.nn.relu(o @ W2_ref[...] + b2_ref[...][None, :])
    o_ref[...] = o @ W3_ref[...] + b3_ref[...][None, :]


def kernel(x, edge_index, Wc1, al1, ar1, bc1, Wc2, al2, ar2, bc2, W1, b1, W2, b2, W3, b3):
    src, dst = edge_index[0], edge_index[1]
    h1 = jax.nn.relu(_gat_conv(x, src, dst, Wc1, al1, ar1, bc1).reshape(N, HEADS * HID))
    h2 = jax.nn.relu(_gat_conv(h1, src, dst, Wc2, al2, ar2, bc2).reshape(N, HEADS * HID))
    hg = jnp.mean(h2, axis=0, keepdims=True)
    o = pl.pallas_call(
        _mlp_kernel,
        out_shape=jax.ShapeDtypeStruct((1, NCLS), jnp.float32),
    )(hg, W1, b1, W2, b2, W3, b3)
    return o


# speedup vs baseline: 26.0454x; 26.0453x over previous
import jax
import jax.numpy as jnp
from jax import lax
from jax.experimental import pallas as pl
from jax.experimental.pallas import tpu as pltpu
from jax.experimental.pallas import tpu_sc as plsc

N = 10000
E = 320000
IN_DIM = 128
HID = 32
HEADS = 8
NCLS = 10

HHALF = HEADS * HID // 2
NSUB = 16
EDGES_PER_SUB = E // NSUB
CHUNK = 80
NCHUNKS = EDGES_PER_SUB // CHUNK
NPAD = 10240
ROWS_PER_SUB = NPAD // NSUB
BN = 1280
NBLK = NPAD // BN


def _proj1_body(x_ref, w_ref, wal_ref, war_ref, f_ref, el_ref, er_ref):
    x = x_ref[...]
    f_ref[...] = jnp.dot(x, w_ref[...], preferred_element_type=jnp.float32)
    el_ref[...] = jnp.dot(x, wal_ref[0], preferred_element_type=jnp.float32)
    er_ref[...] = jnp.dot(x, war_ref[0], preferred_element_type=jnp.float32)


def _proj1(x_pad, w, wal, war):
    f32 = jnp.float32
    return pl.pallas_call(
        _proj1_body,
        grid=(2 * NBLK,),
        in_specs=[pl.BlockSpec((BN, IN_DIM), lambda t: (t % NBLK, 0)),
                  pl.BlockSpec((IN_DIM, HHALF), lambda t: (0, t // NBLK)),
                  pl.BlockSpec((1, IN_DIM, 16), lambda t: (t // NBLK, 0, 0)),
                  pl.BlockSpec((1, IN_DIM, 16), lambda t: (t // NBLK, 0, 0))],
        out_specs=[pl.BlockSpec((BN, HHALF), lambda t: (t, 0)),
                   pl.BlockSpec((BN, 16), lambda t: (t, 0)),
                   pl.BlockSpec((BN, 16), lambda t: (t, 0))],
        out_shape=[jax.ShapeDtypeStruct((2 * NPAD, HHALF), f32),
                   jax.ShapeDtypeStruct((2 * NPAD, 16), f32),
                   jax.ShapeDtypeStruct((2 * NPAD, 16), f32)],
    )(x_pad, w, wal, war)


def _proj2_body(hlo_ref, hhi_ref, w_ref, wal_ref, war_ref,
                f_ref, el_ref, er_ref):
    hlo = hlo_ref[...]
    hhi = hhi_ref[...]
    f_ref[...] = (
        jnp.dot(hlo, w_ref[:HHALF], preferred_element_type=jnp.float32)
        + jnp.dot(hhi, w_ref[HHALF:], preferred_element_type=jnp.float32))
    el_ref[...] = (
        jnp.dot(hlo, wal_ref[0, :HHALF], preferred_element_type=jnp.float32)
        + jnp.dot(hhi, wal_ref[0, HHALF:], preferred_element_type=jnp.float32))
    er_ref[...] = (
        jnp.dot(hlo, war_ref[0, :HHALF], preferred_element_type=jnp.float32)
        + jnp.dot(hhi, war_ref[0, HHALF:], preferred_element_type=jnp.float32))


def _proj2(h, w, wal, war):
    f32 = jnp.float32
    return pl.pallas_call(
        _proj2_body,
        grid=(2 * NBLK,),
        in_specs=[pl.BlockSpec((BN, HHALF), lambda t: (t % NBLK, 0)),
                  pl.BlockSpec((BN, HHALF), lambda t: (NBLK + t % NBLK, 0)),
                  pl.BlockSpec((HEADS * HID, HHALF), lambda t: (0, t // NBLK)),
                  pl.BlockSpec((1, HEADS * HID, 16), lambda t: (t // NBLK, 0, 0)),
                  pl.BlockSpec((1, HEADS * HID, 16), lambda t: (t // NBLK, 0, 0))],
        out_specs=[pl.BlockSpec((BN, HHALF), lambda t: (t, 0)),
                   pl.BlockSpec((BN, 16), lambda t: (t, 0)),
                   pl.BlockSpec((BN, 16), lambda t: (t, 0))],
        out_shape=[jax.ShapeDtypeStruct((2 * NPAD, HHALF), f32),
                   jax.ShapeDtypeStruct((2 * NPAD, 16), f32),
                   jax.ShapeDtypeStruct((2 * NPAD, 16), f32)],
    )(h, h, w, wal, war)


def _sc_edge_body(src_hbm, dst_hbm, f_hbm, el_hbm, er_hbm, b_hbm,
                  out_hbm,
                  sidx_v, didx_v, sadj_v, dadj_v, fbuf_v, rowbuf_v,
                  elbuf_v, erbuf_v, exrow_v, bias_v, acc_feat_sh, acc_ex_sh,
                  sem_f, sem_e):
    c = lax.axis_index("c")
    s = lax.axis_index("s")
    iota = lax.iota(jnp.int32, 16)
    lane_d4 = iota // 4
    lane_m4 = iota % 4
    zero16 = jnp.zeros((16,), jnp.float32)
    coff = jnp.broadcast_to(c * NPAD, (16,)).astype(jnp.int32)

    pltpu.sync_copy(b_hbm, bias_v)

    def _zero_rowbuf(i, carry):
        for j in range(HHALF // 16):
            rowbuf_v[i, pl.ds(j * 16, 16)] = zero16
        exrow_v[i, pl.ds(0, 16)] = zero16
        return carry

    lax.fori_loop(0, CHUNK, _zero_rowbuf, 0)

    rbase = pl.multiple_of(s * ROWS_PER_SUB, CHUNK)
    for k in range(ROWS_PER_SUB // CHUNK):
        r0 = pl.multiple_of(rbase + k * CHUNK, CHUNK)
        pltpu.sync_copy(rowbuf_v, acc_feat_sh.at[pl.ds(r0, CHUNK)])
        pltpu.sync_copy(exrow_v, acc_ex_sh.at[pl.ds(r0, CHUNK)])
    plsc.subcore_barrier()

    ebase = s * EDGES_PER_SUB

    def chunk_body(i, carry):
        off = pl.multiple_of(ebase + i * CHUNK, CHUNK)
        pltpu.sync_copy(src_hbm.at[pl.ds(off, CHUNK)], sidx_v)
        pltpu.sync_copy(dst_hbm.at[pl.ds(off, CHUNK)], didx_v)
        for b in range(CHUNK // 16):
            sl = pl.ds(b * 16, 16)
            sadj_v[sl] = sidx_v[sl] + coff
            dadj_v[sl] = didx_v[sl] + coff
        cp_f = pltpu.async_copy(f_hbm.at[sadj_v], fbuf_v, sem_f)
        cp_el = pltpu.async_copy(el_hbm.at[sadj_v], elbuf_v, sem_e)
        cp_er = pltpu.async_copy(er_hbm.at[dadj_v], erbuf_v, sem_e)
        cp_el.wait()
        cp_er.wait()

        def grp_body(g, carry2):
            pos = g * 4 + lane_d4
            elv = plsc.load_gather(elbuf_v, [pos, lane_m4])
            erv = plsc.load_gather(erbuf_v, [pos, lane_m4])
            e = elv + erv
            e = jnp.where(e >= 0.0, e, 0.2 * e)
            ex = jnp.exp(e)
            plsc.store_scatter(exrow_v, [pos, lane_m4], ex)
            return carry2

        lax.fori_loop(0, CHUNK // 4, grp_body, 0)
        cp_f.wait()

        def edge_body(k2, carry2):
            erow = jnp.broadcast_to(k2, (16,)).astype(jnp.int32)
            for h in range(4):
                hcol = jnp.broadcast_to(jnp.int32(h), (16,))
                m = plsc.load_gather(exrow_v, [erow, hcol])
                for j2 in range(2):
                    col = h * 32 + j2 * 16
                    rowbuf_v[k2, pl.ds(col, 16)] = (
                        fbuf_v[k2, pl.ds(col, 16)] * m)
            return carry2

        lax.fori_loop(0, CHUNK, edge_body, 0)

        pltpu.sync_copy(rowbuf_v, acc_feat_sh.at[didx_v], add=True)
        pltpu.sync_copy(exrow_v, acc_ex_sh.at[didx_v], add=True)
        return carry

    lax.fori_loop(0, NCHUNKS, chunk_body, 0)
    plsc.subcore_barrier()

    obase = pl.multiple_of(c * NPAD + rbase, CHUNK)

    def fin_body(k, carry):
        r0 = pl.multiple_of(rbase + k * CHUNK, CHUNK)
        o0 = pl.multiple_of(obase + k * CHUNK, CHUNK)
        pltpu.sync_copy(acc_feat_sh.at[pl.ds(r0, CHUNK)], fbuf_v)
        pltpu.sync_copy(acc_ex_sh.at[pl.ds(r0, CHUNK)], exrow_v)

        def row_body(r, carry2):
            rr = jnp.broadcast_to(r, (16,)).astype(jnp.int32)
            for h in range(4):
                hcol = jnp.broadcast_to(jnp.int32(h), (16,))
                d = plsc.load_gather(exrow_v, [rr, hcol])
                d = jnp.where(d > 0.0, d, 1.0)
                for j2 in range(2):
                    col = h * 32 + j2 * 16
                    v = fbuf_v[r, pl.ds(col, 16)] / d + bias_v[c, pl.ds(col, 16)]
                    rowbuf_v[r, pl.ds(col, 16)] = jnp.maximum(v, 0.0)
            return carry2

        lax.fori_loop(0, CHUNK, row_body, 0)
        pltpu.sync_copy(rowbuf_v, out_hbm.at[pl.ds(o0, CHUNK)])
        return carry

    lax.fori_loop(0, ROWS_PER_SUB // CHUNK, fin_body, 0)


def _sc_edge(src, dst, f_all, el_all, er_all, b_all):
    f32 = jnp.float32
    call = pl.kernel(
        _sc_edge_body,
        out_type=jax.ShapeDtypeStruct((2 * NPAD, HHALF), f32),
        mesh=plsc.VectorSubcoreMesh(core_axis_name="c", subcore_axis_name="s"),
        scratch_types=(
            pltpu.VMEM((CHUNK,), jnp.int32),
            pltpu.VMEM((CHUNK,), jnp.int32),
            pltpu.VMEM((CHUNK,), jnp.int32),
            pltpu.VMEM((CHUNK,), jnp.int32),
            pltpu.VMEM((CHUNK, HHALF), f32),
            pltpu.VMEM((CHUNK, HHALF), f32),
            pltpu.VMEM((CHUNK, 16), f32),
            pltpu.VMEM((CHUNK, 16), f32),
            pltpu.VMEM((CHUNK, 16), f32),
            pltpu.VMEM((2, HHALF), f32),
            pltpu.VMEM_SHARED((NPAD, HHALF), f32),
            pltpu.VMEM_SHARED((NPAD, 16), f32),
            pltpu.SemaphoreType.DMA,
            pltpu.SemaphoreType.DMA,
        ),
        compiler_params=pltpu.CompilerParams(use_tc_tiling_on_sc=False, needs_layout_passes=False),
    )
    return call(src, dst, f_all, el_all, er_all, b_all)


def _head_body(h_ref, w1_ref, b1_ref, w2_ref, b2_ref, w3_ref, b3_ref,
               o_ref, acc_ref):
    t = pl.program_id(0)
    g = t // NBLK
    i = t % NBLK

    @pl.when(t == 0)
    def _():
        acc_ref[...] = jnp.zeros_like(acc_ref)

    rows = jax.lax.broadcasted_iota(jnp.int32, (BN, 1), 0) + i * BN
    blk = jnp.where(rows < N, h_ref[...], 0.0)
    acc_ref[0:1, pl.ds(g * HHALF, HHALF)] += jnp.sum(blk, axis=0, keepdims=True)

    @pl.when(t == 2 * NBLK - 1)
    def _():
        hg = acc_ref[...] * (1.0 / N)
        o = jnp.dot(hg, w1_ref[...], preferred_element_type=jnp.float32)
        o = jnp.maximum(o + b1_ref[...], 0.0)
        o = jnp.maximum(jnp.dot(o, w2_ref[...], preferred_element_type=jnp.float32)
                        + b2_ref[...], 0.0)
        o_ref[...] = (jnp.dot(o, w3_ref[...], preferred_element_type=jnp.float32)
                      + b3_ref[...])


def _full(shape):
    return pl.BlockSpec(shape, lambda t: tuple(0 for _ in shape))


def _head(h, w1, b1, w2, b2, w3, b3):
    return pl.pallas_call(
        _head_body,
        grid=(2 * NBLK,),
        in_specs=[pl.BlockSpec((BN, HHALF), lambda t: (t, 0)),
                  _full((HEADS * HID, HID)),
                  _full((1, HID)),
                  _full((HID, HID)),
                  _full((1, HID)),
                  _full((HID, NCLS)),
                  _full((1, NCLS))],
        out_specs=pl.BlockSpec((1, NCLS), lambda t: (0, 0)),
        out_shape=jax.ShapeDtypeStruct((1, NCLS), jnp.float32),
        scratch_shapes=[pltpu.VMEM((1, HEADS * HID), jnp.float32)],
    )(h, w1, b1.reshape(1, HID), w2, b2.reshape(1, HID), w3,
      b3.reshape(1, NCLS))


def _attn_matrix(a, h0):
    sel = jnp.eye(HEADS, 16, k=-h0, dtype=a.dtype)
    return (sel[:, None, :] * a[:, :, None]).reshape(HEADS * HID, 16)


def _logit_weights(wc, al, ar):
    wal = jnp.stack([wc @ _attn_matrix(al, 0), wc @ _attn_matrix(al, 4)])
    war = jnp.stack([wc @ _attn_matrix(ar, 0), wc @ _attn_matrix(ar, 4)])
    return wal, war


def kernel(x, edge_index, Wc1, al1, ar1, bc1, Wc2, al2, ar2, bc2,
           W1, b1, W2, b2, W3, b3):
    src = edge_index[0]
    dst = edge_index[1]
    x_pad = jnp.pad(x, ((0, NPAD - N), (0, 0)))

    wal1, war1 = _logit_weights(Wc1, al1, ar1)
    f1, el1, er1 = _proj1(x_pad, Wc1, wal1, war1)
    h1 = _sc_edge(src, dst, f1, el1, er1, bc1.reshape(2, HHALF))

    wal2, war2 = _logit_weights(Wc2, al2, ar2)
    f2, el2, er2 = _proj2(h1, Wc2, wal2, war2)
    h2 = _sc_edge(src, dst, f2, el2, er2, bc2.reshape(2, HHALF))

    return _head(h2, W1, b1, W2, b2, W3, b3)


# speedup vs baseline: 51.2562x; 1.9680x over previous
import jax
import jax.numpy as jnp
from jax import lax
from jax.experimental import pallas as pl
from jax.experimental.pallas import tpu as pltpu
from jax.experimental.pallas import tpu_sc as plsc

N = 10000
E = 320000
IN_DIM = 128
HID = 32
HEADS = 8
NCLS = 10

HHALF = HEADS * HID // 2
NSUB = 16
EDGES_PER_SUB = E // NSUB
CHUNK = 80
NCHUNKS = EDGES_PER_SUB // CHUNK
NPAD = 10240
ROWS_PER_SUB = NPAD // NSUB
BN = 1280
NBLK = NPAD // BN


def _proj1_body(x_ref, w_ref, wal_ref, war_ref, f_ref, el_ref, er_ref):
    x = x_ref[...]
    f_ref[...] = jnp.dot(x, w_ref[...], preferred_element_type=jnp.float32)
    el_ref[...] = jnp.dot(x, wal_ref[0], preferred_element_type=jnp.float32)
    er_ref[...] = jnp.dot(x, war_ref[0], preferred_element_type=jnp.float32)


def _proj1(x_pad, w, wal, war):
    f32 = jnp.float32
    return pl.pallas_call(
        _proj1_body,
        grid=(2 * NBLK,),
        in_specs=[pl.BlockSpec((BN, IN_DIM), lambda t: (t % NBLK, 0)),
                  pl.BlockSpec((IN_DIM, HHALF), lambda t: (0, t // NBLK)),
                  pl.BlockSpec((1, IN_DIM, 16), lambda t: (t // NBLK, 0, 0)),
                  pl.BlockSpec((1, IN_DIM, 16), lambda t: (t // NBLK, 0, 0))],
        out_specs=[pl.BlockSpec((BN, HHALF), lambda t: (t, 0)),
                   pl.BlockSpec((BN, 16), lambda t: (t, 0)),
                   pl.BlockSpec((BN, 16), lambda t: (t, 0))],
        out_shape=[jax.ShapeDtypeStruct((2 * NPAD, HHALF), f32),
                   jax.ShapeDtypeStruct((2 * NPAD, 16), f32),
                   jax.ShapeDtypeStruct((2 * NPAD, 16), f32)],
    )(x_pad, w, wal, war)


def _proj2_body(hlo_ref, hhi_ref, w_ref, wal_ref, war_ref,
                f_ref, el_ref, er_ref):
    hlo = hlo_ref[...]
    hhi = hhi_ref[...]
    f_ref[...] = (
        jnp.dot(hlo, w_ref[:HHALF], preferred_element_type=jnp.float32)
        + jnp.dot(hhi, w_ref[HHALF:], preferred_element_type=jnp.float32))
    el_ref[...] = (
        jnp.dot(hlo, wal_ref[0, :HHALF], preferred_element_type=jnp.float32)
        + jnp.dot(hhi, wal_ref[0, HHALF:], preferred_element_type=jnp.float32))
    er_ref[...] = (
        jnp.dot(hlo, war_ref[0, :HHALF], preferred_element_type=jnp.float32)
        + jnp.dot(hhi, war_ref[0, HHALF:], preferred_element_type=jnp.float32))


def _proj2(h, w, wal, war):
    f32 = jnp.float32
    return pl.pallas_call(
        _proj2_body,
        grid=(2 * NBLK,),
        in_specs=[pl.BlockSpec((BN, HHALF), lambda t: (t % NBLK, 0)),
                  pl.BlockSpec((BN, HHALF), lambda t: (NBLK + t % NBLK, 0)),
                  pl.BlockSpec((HEADS * HID, HHALF), lambda t: (0, t // NBLK)),
                  pl.BlockSpec((1, HEADS * HID, 16), lambda t: (t // NBLK, 0, 0)),
                  pl.BlockSpec((1, HEADS * HID, 16), lambda t: (t // NBLK, 0, 0))],
        out_specs=[pl.BlockSpec((BN, HHALF), lambda t: (t, 0)),
                   pl.BlockSpec((BN, 16), lambda t: (t, 0)),
                   pl.BlockSpec((BN, 16), lambda t: (t, 0))],
        out_shape=[jax.ShapeDtypeStruct((2 * NPAD, HHALF), f32),
                   jax.ShapeDtypeStruct((2 * NPAD, 16), f32),
                   jax.ShapeDtypeStruct((2 * NPAD, 16), f32)],
    )(h, h, w, wal, war)


def _sc_edge_body(src_hbm, dst_hbm, f_hbm, el_hbm, er_hbm, b_hbm,
                  out_hbm,
                  sidx0, sidx1, didx0, didx1, sadj0, sadj1, dadj0, dadj1,
                  rawd0, rawd1, fb0, fb1, elb0, elb1, erb0, erb1, exr0, exr1,
                  bias_v, acc_feat_sh, acc_ex_sh,
                  sf0, sf1, se0, se1, si0, si1, swf0, swf1, swe0, swe1):
    c = lax.axis_index("c")
    s = lax.axis_index("s")
    iota = lax.iota(jnp.int32, 16)
    lane_d4 = iota // 4
    lane_m4 = iota % 4
    zero16 = jnp.zeros((16,), jnp.float32)
    coff = jnp.broadcast_to(c * NPAD, (16,)).astype(jnp.int32)

    SIDX = (sidx0, sidx1)
    DIDX = (didx0, didx1)
    SADJ = (sadj0, sadj1)
    DADJ = (dadj0, dadj1)
    RAWD = (rawd0, rawd1)
    FB = (fb0, fb1)
    ELB = (elb0, elb1)
    ERB = (erb0, erb1)
    EXR = (exr0, exr1)
    SF = (sf0, sf1)
    SE = (se0, se1)
    SI = (si0, si1)
    SWF = (swf0, swf1)
    SWE = (swe0, swe1)

    pltpu.sync_copy(b_hbm, bias_v)

    def _zero_bufs(i, carry):
        for j in range(HHALF // 16):
            fb0[i, pl.ds(j * 16, 16)] = zero16
        exr0[i, pl.ds(0, 16)] = zero16
        exr1[i, pl.ds(0, 16)] = zero16
        return carry

    lax.fori_loop(0, CHUNK, _zero_bufs, 0)

    rbase = pl.multiple_of(s * ROWS_PER_SUB, CHUNK)
    for k in range(ROWS_PER_SUB // CHUNK):
        r0 = pl.multiple_of(rbase + k * CHUNK, CHUNK)
        pltpu.sync_copy(fb0, acc_feat_sh.at[pl.ds(r0, CHUNK)])
        pltpu.sync_copy(exr0, acc_ex_sh.at[pl.ds(r0, CHUNK)])
    plsc.subcore_barrier()

    ebase = s * EDGES_PER_SUB

    def chunk_off(i):
        off = jnp.minimum(ebase + i * CHUNK, E - CHUNK)
        return pl.multiple_of(off, 8)

    def issue_idx(i, j):
        pltpu.async_copy(src_hbm.at[pl.ds(chunk_off(i), CHUNK)], SIDX[j], SI[j])
        pltpu.async_copy(dst_hbm.at[pl.ds(chunk_off(i), CHUNK)], DIDX[j], SI[j])

    def wait_idx(j):
        pltpu.make_async_copy(src_hbm.at[pl.ds(0, CHUNK)], SIDX[j], SI[j]).wait()
        pltpu.make_async_copy(dst_hbm.at[pl.ds(0, CHUNK)], DIDX[j], SI[j]).wait()

    def adjust_idx(j):
        for b in range(CHUNK // 16):
            sl = pl.ds(b * 16, 16)
            sv = SIDX[j][sl]
            dv = DIDX[j][sl]
            SADJ[j][sl] = sv + coff
            DADJ[j][sl] = dv + coff
            RAWD[j][sl] = dv

    def issue_gathers(j):
        pltpu.async_copy(f_hbm.at[SADJ[j]], FB[j], SF[j])
        pltpu.async_copy(el_hbm.at[SADJ[j]], ELB[j], SE[j])
        pltpu.async_copy(er_hbm.at[DADJ[j]], ERB[j], SE[j])

    def wait_gathers(j):
        pltpu.make_async_copy(f_hbm.at[pl.ds(0, CHUNK)], FB[j], SF[j]).wait()
        pltpu.make_async_copy(el_hbm.at[pl.ds(0, CHUNK)], ELB[j], SE[j]).wait()
        pltpu.make_async_copy(er_hbm.at[pl.ds(0, CHUNK)], ERB[j], SE[j]).wait()

    def wait_scatters(j):
        pltpu.make_async_copy(
            FB[j], acc_feat_sh.at[pl.ds(0, CHUNK)], SWF[j]).wait()
        pltpu.make_async_copy(
            EXR[j], acc_ex_sh.at[pl.ds(0, CHUNK)], SWE[j]).wait()

    def compute_and_scatter(j):
        def grp_body(g, carry2):
            pos = g * 4 + lane_d4
            elv = plsc.load_gather(ELB[j], [pos, lane_m4])
            erv = plsc.load_gather(ERB[j], [pos, lane_m4])
            e = elv + erv
            e = jnp.where(e >= 0.0, e, 0.2 * e)
            ex = jnp.exp(e)
            plsc.store_scatter(EXR[j], [pos, lane_m4], ex)
            return carry2

        lax.fori_loop(0, CHUNK // 4, grp_body, 0)

        def edge_body(k2, carry2):
            erow = jnp.broadcast_to(k2, (16,)).astype(jnp.int32)
            for h in range(4):
                hcol = jnp.broadcast_to(jnp.int32(h), (16,))
                m = plsc.load_gather(EXR[j], [erow, hcol])
                for j2 in range(2):
                    col = h * 32 + j2 * 16
                    FB[j][k2, pl.ds(col, 16)] = FB[j][k2, pl.ds(col, 16)] * m
            return carry2

        lax.fori_loop(0, CHUNK, edge_body, 0)
        pltpu.async_copy(FB[j], acc_feat_sh.at[RAWD[j]], SWF[j], add=True)
        pltpu.async_copy(EXR[j], acc_ex_sh.at[RAWD[j]], SWE[j], add=True)

    issue_idx(0, 0)
    issue_idx(1, 1)
    wait_idx(0)
    adjust_idx(0)
    issue_gathers(0)
    wait_idx(1)
    adjust_idx(1)
    issue_gathers(1)
    wait_gathers(0)
    compute_and_scatter(0)
    issue_idx(2, 0)

    def pair_body(p, carry):
        for j, q in ((1, 0), (0, 1)):
            i = 1 + 2 * p + (1 - j)
            wait_idx(q)
            wait_scatters(q)
            adjust_idx(q)
            issue_gathers(q)
            wait_gathers(j)
            compute_and_scatter(j)
            issue_idx(i + 2, j)
        return carry

    lax.fori_loop(0, (NCHUNKS - 2) // 2, pair_body, 0)

    wait_gathers(1)
    compute_and_scatter(1)
    wait_idx(0)
    wait_scatters(0)
    wait_scatters(1)
    plsc.subcore_barrier()

    obase = pl.multiple_of(c * NPAD + rbase, CHUNK)

    def fin_body(k, carry):
        r0 = pl.multiple_of(rbase + k * CHUNK, CHUNK)
        o0 = pl.multiple_of(obase + k * CHUNK, CHUNK)
        pltpu.sync_copy(acc_feat_sh.at[pl.ds(r0, CHUNK)], fb0)
        pltpu.sync_copy(acc_ex_sh.at[pl.ds(r0, CHUNK)], exr0)

        def row_body(r, carry2):
            rr = jnp.broadcast_to(r, (16,)).astype(jnp.int32)
            for h in range(4):
                hcol = jnp.broadcast_to(jnp.int32(h), (16,))
                d = plsc.load_gather(exr0, [rr, hcol])
                d = jnp.where(d > 0.0, d, 1.0)
                for j2 in range(2):
                    col = h * 32 + j2 * 16
                    v = fb0[r, pl.ds(col, 16)] / d + bias_v[c, pl.ds(col, 16)]
                    fb1[r, pl.ds(col, 16)] = jnp.maximum(v, 0.0)
            return carry2

        lax.fori_loop(0, CHUNK, row_body, 0)
        pltpu.sync_copy(fb1, out_hbm.at[pl.ds(o0, CHUNK)])
        return carry

    lax.fori_loop(0, ROWS_PER_SUB // CHUNK, fin_body, 0)


def _sc_edge(src, dst, f_all, el_all, er_all, b_all):
    f32 = jnp.float32
    call = pl.kernel(
        _sc_edge_body,
        out_type=jax.ShapeDtypeStruct((2 * NPAD, HHALF), f32),
        mesh=plsc.VectorSubcoreMesh(core_axis_name="c", subcore_axis_name="s"),
        scratch_types=(
            pltpu.VMEM((CHUNK,), jnp.int32),
            pltpu.VMEM((CHUNK,), jnp.int32),
            pltpu.VMEM((CHUNK,), jnp.int32),
            pltpu.VMEM((CHUNK,), jnp.int32),
            pltpu.VMEM((CHUNK,), jnp.int32),
            pltpu.VMEM((CHUNK,), jnp.int32),
            pltpu.VMEM((CHUNK,), jnp.int32),
            pltpu.VMEM((CHUNK,), jnp.int32),
            pltpu.VMEM((CHUNK,), jnp.int32),
            pltpu.VMEM((CHUNK,), jnp.int32),
            pltpu.VMEM((CHUNK, HHALF), f32),
            pltpu.VMEM((CHUNK, HHALF), f32),
            pltpu.VMEM((CHUNK, 16), f32),
            pltpu.VMEM((CHUNK, 16), f32),
            pltpu.VMEM((CHUNK, 16), f32),
            pltpu.VMEM((CHUNK, 16), f32),
            pltpu.VMEM((CHUNK, 16), f32),
            pltpu.VMEM((CHUNK, 16), f32),
            pltpu.VMEM((2, HHALF), f32),
            pltpu.VMEM_SHARED((NPAD, HHALF), f32),
            pltpu.VMEM_SHARED((NPAD, 16), f32),
            pltpu.SemaphoreType.DMA,
            pltpu.SemaphoreType.DMA,
            pltpu.SemaphoreType.DMA,
            pltpu.SemaphoreType.DMA,
            pltpu.SemaphoreType.DMA,
            pltpu.SemaphoreType.DMA,
            pltpu.SemaphoreType.DMA,
            pltpu.SemaphoreType.DMA,
            pltpu.SemaphoreType.DMA,
            pltpu.SemaphoreType.DMA,
        ),
        compiler_params=pltpu.CompilerParams(use_tc_tiling_on_sc=False, needs_layout_passes=False),
    )
    return call(src, dst, f_all, el_all, er_all, b_all)


def _head_body(h_ref, w1_ref, b1_ref, w2_ref, b2_ref, w3_ref, b3_ref,
               o_ref, acc_ref):
    t = pl.program_id(0)
    g = t // NBLK
    i = t % NBLK

    @pl.when(t == 0)
    def _():
        acc_ref[...] = jnp.zeros_like(acc_ref)

    rows = jax.lax.broadcasted_iota(jnp.int32, (BN, 1), 0) + i * BN
    blk = jnp.where(rows < N, h_ref[...], 0.0)
    acc_ref[0:1, pl.ds(g * HHALF, HHALF)] += jnp.sum(blk, axis=0, keepdims=True)

    @pl.when(t == 2 * NBLK - 1)
    def _():
        hg = acc_ref[...] * (1.0 / N)
        o = jnp.dot(hg, w1_ref[...], preferred_element_type=jnp.float32)
        o = jnp.maximum(o + b1_ref[...], 0.0)
        o = jnp.maximum(jnp.dot(o, w2_ref[...], preferred_element_type=jnp.float32)
                        + b2_ref[...], 0.0)
        o_ref[...] = (jnp.dot(o, w3_ref[...], preferred_element_type=jnp.float32)
                      + b3_ref[...])


def _full(shape):
    return pl.BlockSpec(shape, lambda t: tuple(0 for _ in shape))


def _head(h, w1, b1, w2, b2, w3, b3):
    return pl.pallas_call(
        _head_body,
        grid=(2 * NBLK,),
        in_specs=[pl.BlockSpec((BN, HHALF), lambda t: (t, 0)),
                  _full((HEADS * HID, HID)),
                  _full((1, HID)),
                  _full((HID, HID)),
                  _full((1, HID)),
                  _full((HID, NCLS)),
                  _full((1, NCLS))],
        out_specs=pl.BlockSpec((1, NCLS), lambda t: (0, 0)),
        out_shape=jax.ShapeDtypeStruct((1, NCLS), jnp.float32),
        scratch_shapes=[pltpu.VMEM((1, HEADS * HID), jnp.float32)],
    )(h, w1, b1.reshape(1, HID), w2, b2.reshape(1, HID), w3,
      b3.reshape(1, NCLS))


def _attn_matrix(a, h0):
    sel = jnp.eye(HEADS, 16, k=-h0, dtype=a.dtype)
    return (sel[:, None, :] * a[:, :, None]).reshape(HEADS * HID, 16)


def _logit_weights(wc, al, ar):
    wal = jnp.stack([wc @ _attn_matrix(al, 0), wc @ _attn_matrix(al, 4)])
    war = jnp.stack([wc @ _attn_matrix(ar, 0), wc @ _attn_matrix(ar, 4)])
    return wal, war


def kernel(x, edge_index, Wc1, al1, ar1, bc1, Wc2, al2, ar2, bc2,
           W1, b1, W2, b2, W3, b3):
    src = edge_index[0]
    dst = edge_index[1]
    x_pad = jnp.pad(x, ((0, NPAD - N), (0, 0)))

    wal1, war1 = _logit_weights(Wc1, al1, ar1)
    f1, el1, er1 = _proj1(x_pad, Wc1, wal1, war1)
    h1 = _sc_edge(src, dst, f1, el1, er1, bc1.reshape(2, HHALF))

    wal2, war2 = _logit_weights(Wc2, al2, ar2)
    f2, el2, er2 = _proj2(h1, Wc2, wal2, war2)
    h2 = _sc_edge(src, dst, f2, el2, er2, bc2.reshape(2, HHALF))

    return _head(h2, W1, b1, W2, b2, W3, b3)


# speedup vs baseline: 51.3879x; 1.0026x over previous
import jax
import jax.numpy as jnp
from jax import lax
from jax.experimental import pallas as pl
from jax.experimental.pallas import tpu as pltpu
from jax.experimental.pallas import tpu_sc as plsc

N = 10000
E = 320000
IN_DIM = 128
HID = 32
HEADS = 8
NCLS = 10

HHALF = HEADS * HID // 2
NSUB = 16
EDGES_PER_SUB = E // NSUB
CHUNK = 80
NCHUNKS = EDGES_PER_SUB // CHUNK
NPAD = 10240
ROWS_PER_SUB = NPAD // NSUB
BN = 1280
NBLK = NPAD // BN


def _proj1_body(x_ref, w_ref, wal_ref, war_ref, f_ref, el_ref, er_ref):
    x = x_ref[...]
    f_ref[...] = jnp.dot(x, w_ref[...], preferred_element_type=jnp.float32)
    el_ref[...] = jnp.dot(x, wal_ref[0], preferred_element_type=jnp.float32)
    er_ref[...] = jnp.dot(x, war_ref[0], preferred_element_type=jnp.float32)


def _proj1(x_pad, w, wal, war):
    f32 = jnp.float32
    return pl.pallas_call(
        _proj1_body,
        grid=(2 * NBLK,),
        in_specs=[pl.BlockSpec((BN, IN_DIM), lambda t: (t % NBLK, 0)),
                  pl.BlockSpec((IN_DIM, HHALF), lambda t: (0, t // NBLK)),
                  pl.BlockSpec((1, IN_DIM, 16), lambda t: (t // NBLK, 0, 0)),
                  pl.BlockSpec((1, IN_DIM, 16), lambda t: (t // NBLK, 0, 0))],
        out_specs=[pl.BlockSpec((BN, HHALF), lambda t: (t, 0)),
                   pl.BlockSpec((BN, 16), lambda t: (t, 0)),
                   pl.BlockSpec((BN, 16), lambda t: (t, 0))],
        out_shape=[jax.ShapeDtypeStruct((2 * NPAD, HHALF), f32),
                   jax.ShapeDtypeStruct((2 * NPAD, 16), f32),
                   jax.ShapeDtypeStruct((2 * NPAD, 16), f32)],
    )(x_pad, w, wal, war)


def _proj2_body(hlo_ref, hhi_ref, w_ref, wal_ref, war_ref,
                f_ref, el_ref, er_ref):
    hlo = hlo_ref[...]
    hhi = hhi_ref[...]
    f_ref[...] = (
        jnp.dot(hlo, w_ref[:HHALF], preferred_element_type=jnp.float32)
        + jnp.dot(hhi, w_ref[HHALF:], preferred_element_type=jnp.float32))
    el_ref[...] = (
        jnp.dot(hlo, wal_ref[0, :HHALF], preferred_element_type=jnp.float32)
        + jnp.dot(hhi, wal_ref[0, HHALF:], preferred_element_type=jnp.float32))
    er_ref[...] = (
        jnp.dot(hlo, war_ref[0, :HHALF], preferred_element_type=jnp.float32)
        + jnp.dot(hhi, war_ref[0, HHALF:], preferred_element_type=jnp.float32))


def _proj2(h, w, wal, war):
    f32 = jnp.float32
    return pl.pallas_call(
        _proj2_body,
        grid=(2 * NBLK,),
        in_specs=[pl.BlockSpec((BN, HHALF), lambda t: (t % NBLK, 0)),
                  pl.BlockSpec((BN, HHALF), lambda t: (NBLK + t % NBLK, 0)),
                  pl.BlockSpec((HEADS * HID, HHALF), lambda t: (0, t // NBLK)),
                  pl.BlockSpec((1, HEADS * HID, 16), lambda t: (t // NBLK, 0, 0)),
                  pl.BlockSpec((1, HEADS * HID, 16), lambda t: (t // NBLK, 0, 0))],
        out_specs=[pl.BlockSpec((BN, HHALF), lambda t: (t, 0)),
                   pl.BlockSpec((BN, 16), lambda t: (t, 0)),
                   pl.BlockSpec((BN, 16), lambda t: (t, 0))],
        out_shape=[jax.ShapeDtypeStruct((2 * NPAD, HHALF), f32),
                   jax.ShapeDtypeStruct((2 * NPAD, 16), f32),
                   jax.ShapeDtypeStruct((2 * NPAD, 16), f32)],
    )(h, h, w, wal, war)


def _sc_edge_body(src_hbm, dst_hbm, f_hbm, el_hbm, er_hbm, b_hbm,
                  out_hbm,
                  sidx0, sidx1, didx0, didx1, sadj0, sadj1, dadj0, dadj1,
                  rawd0, rawd1, fb0, fb1, elb0, elb1, erb0, erb1, exr0, exr1,
                  bias_v, acc_feat_sh, acc_ex_sh,
                  sf0, sf1, se0, se1, si0, si1, swf0, swf1, swe0, swe1):
    c = lax.axis_index("c")
    s = lax.axis_index("s")
    iota = lax.iota(jnp.int32, 16)
    lane_d4 = iota // 4
    lane_m4 = iota % 4
    zero16 = jnp.zeros((16,), jnp.float32)
    coff = jnp.broadcast_to(c * NPAD, (16,)).astype(jnp.int32)

    SIDX = (sidx0, sidx1)
    DIDX = (didx0, didx1)
    SADJ = (sadj0, sadj1)
    DADJ = (dadj0, dadj1)
    RAWD = (rawd0, rawd1)
    FB = (fb0, fb1)
    ELB = (elb0, elb1)
    ERB = (erb0, erb1)
    EXR = (exr0, exr1)
    SF = (sf0, sf1)
    SE = (se0, se1)
    SI = (si0, si1)
    SWF = (swf0, swf1)
    SWE = (swe0, swe1)

    pltpu.sync_copy(b_hbm, bias_v)

    def _zero_bufs(i, carry):
        for j in range(HHALF // 16):
            fb0[i, pl.ds(j * 16, 16)] = zero16
        exr0[i, pl.ds(0, 16)] = zero16
        exr1[i, pl.ds(0, 16)] = zero16
        return carry

    lax.fori_loop(0, CHUNK, _zero_bufs, 0)

    rbase = pl.multiple_of(s * ROWS_PER_SUB, CHUNK)
    for k in range(ROWS_PER_SUB // CHUNK):
        r0 = pl.multiple_of(rbase + k * CHUNK, CHUNK)
        pltpu.sync_copy(fb0, acc_feat_sh.at[pl.ds(r0, CHUNK)])
        pltpu.sync_copy(exr0, acc_ex_sh.at[pl.ds(r0, CHUNK)])
    plsc.subcore_barrier()

    ebase = s * EDGES_PER_SUB

    def chunk_off(i):
        off = jnp.minimum(ebase + i * CHUNK, E - CHUNK)
        return pl.multiple_of(off, 8)

    def issue_idx(i, j):
        pltpu.async_copy(src_hbm.at[pl.ds(chunk_off(i), CHUNK)], SIDX[j], SI[j])
        pltpu.async_copy(dst_hbm.at[pl.ds(chunk_off(i), CHUNK)], DIDX[j], SI[j])

    def wait_idx(j):
        pltpu.make_async_copy(src_hbm.at[pl.ds(0, CHUNK)], SIDX[j], SI[j]).wait()
        pltpu.make_async_copy(dst_hbm.at[pl.ds(0, CHUNK)], DIDX[j], SI[j]).wait()

    def adjust_idx(j):
        for b in range(CHUNK // 16):
            sl = pl.ds(b * 16, 16)
            sv = SIDX[j][sl]
            dv = DIDX[j][sl]
            SADJ[j][sl] = sv + coff
            DADJ[j][sl] = dv + coff
            RAWD[j][sl] = dv

    def issue_gathers(j):
        pltpu.async_copy(f_hbm.at[SADJ[j]], FB[j], SF[j])
        pltpu.async_copy(el_hbm.at[SADJ[j]], ELB[j], SE[j])
        pltpu.async_copy(er_hbm.at[DADJ[j]], ERB[j], SE[j])

    def wait_gathers(j):
        pltpu.make_async_copy(f_hbm.at[pl.ds(0, CHUNK)], FB[j], SF[j]).wait()
        pltpu.make_async_copy(el_hbm.at[pl.ds(0, CHUNK)], ELB[j], SE[j]).wait()
        pltpu.make_async_copy(er_hbm.at[pl.ds(0, CHUNK)], ERB[j], SE[j]).wait()

    def wait_scatters(j):
        pltpu.make_async_copy(
            FB[j], acc_feat_sh.at[pl.ds(0, CHUNK)], SWF[j]).wait()
        pltpu.make_async_copy(
            EXR[j], acc_ex_sh.at[pl.ds(0, CHUNK)], SWE[j]).wait()

    def compute_and_scatter(j):
        def grp_body(g2, carry2):
            for u in range(2):
                pos = (g2 * 2 + u) * 4 + lane_d4
                elv = plsc.load_gather(ELB[j], [pos, lane_m4])
                erv = plsc.load_gather(ERB[j], [pos, lane_m4])
                e = elv + erv
                e = jnp.where(e >= 0.0, e, 0.2 * e)
                ex = jnp.exp(e)
                plsc.store_scatter(EXR[j], [pos, lane_m4], ex)
            return carry2

        lax.fori_loop(0, CHUNK // 8, grp_body, 0)

        def edge_body(g, carry2):
            for u in range(4):
                k2 = g * 4 + u
                erow = jnp.broadcast_to(k2, (16,)).astype(jnp.int32)
                for h in range(4):
                    hcol = jnp.broadcast_to(jnp.int32(h), (16,))
                    m = plsc.load_gather(EXR[j], [erow, hcol])
                    for j2 in range(2):
                        col = h * 32 + j2 * 16
                        FB[j][k2, pl.ds(col, 16)] = (
                            FB[j][k2, pl.ds(col, 16)] * m)
            return carry2

        lax.fori_loop(0, CHUNK // 4, edge_body, 0)
        pltpu.async_copy(FB[j], acc_feat_sh.at[RAWD[j]], SWF[j], add=True)
        pltpu.async_copy(EXR[j], acc_ex_sh.at[RAWD[j]], SWE[j], add=True)

    issue_idx(0, 0)
    issue_idx(1, 1)
    wait_idx(0)
    adjust_idx(0)
    issue_gathers(0)
    wait_idx(1)
    adjust_idx(1)
    issue_gathers(1)
    wait_gathers(0)
    compute_and_scatter(0)
    issue_idx(2, 0)

    def pair_body(p, carry):
        for j, q in ((1, 0), (0, 1)):
            i = 1 + 2 * p + (1 - j)
            wait_idx(q)
            wait_scatters(q)
            adjust_idx(q)
            issue_gathers(q)
            wait_gathers(j)
            compute_and_scatter(j)
            issue_idx(i + 2, j)
        return carry

    lax.fori_loop(0, (NCHUNKS - 2) // 2, pair_body, 0)

    wait_gathers(1)
    compute_and_scatter(1)
    wait_idx(0)
    wait_scatters(0)
    wait_scatters(1)
    plsc.subcore_barrier()

    obase = pl.multiple_of(c * NPAD + rbase, CHUNK)

    def fin_body(k, carry):
        r0 = pl.multiple_of(rbase + k * CHUNK, CHUNK)
        o0 = pl.multiple_of(obase + k * CHUNK, CHUNK)
        pltpu.sync_copy(acc_feat_sh.at[pl.ds(r0, CHUNK)], fb0)
        pltpu.sync_copy(acc_ex_sh.at[pl.ds(r0, CHUNK)], exr0)

        def row_body(r, carry2):
            rr = jnp.broadcast_to(r, (16,)).astype(jnp.int32)
            for h in range(4):
                hcol = jnp.broadcast_to(jnp.int32(h), (16,))
                d = plsc.load_gather(exr0, [rr, hcol])
                d = jnp.where(d > 0.0, d, 1.0)
                for j2 in range(2):
                    col = h * 32 + j2 * 16
                    v = fb0[r, pl.ds(col, 16)] / d + bias_v[c, pl.ds(col, 16)]
                    fb1[r, pl.ds(col, 16)] = jnp.maximum(v, 0.0)
            return carry2

        lax.fori_loop(0, CHUNK, row_body, 0)
        pltpu.sync_copy(fb1, out_hbm.at[pl.ds(o0, CHUNK)])
        return carry

    lax.fori_loop(0, ROWS_PER_SUB // CHUNK, fin_body, 0)


def _sc_edge(src, dst, f_all, el_all, er_all, b_all):
    f32 = jnp.float32
    call = pl.kernel(
        _sc_edge_body,
        out_type=jax.ShapeDtypeStruct((2 * NPAD, HHALF), f32),
        mesh=plsc.VectorSubcoreMesh(core_axis_name="c", subcore_axis_name="s"),
        scratch_types=(
            pltpu.VMEM((CHUNK,), jnp.int32),
            pltpu.VMEM((CHUNK,), jnp.int32),
            pltpu.VMEM((CHUNK,), jnp.int32),
            pltpu.VMEM((CHUNK,), jnp.int32),
            pltpu.VMEM((CHUNK,), jnp.int32),
            pltpu.VMEM((CHUNK,), jnp.int32),
            pltpu.VMEM((CHUNK,), jnp.int32),
            pltpu.VMEM((CHUNK,), jnp.int32),
            pltpu.VMEM((CHUNK,), jnp.int32),
            pltpu.VMEM((CHUNK,), jnp.int32),
            pltpu.VMEM((CHUNK, HHALF), f32),
            pltpu.VMEM((CHUNK, HHALF), f32),
            pltpu.VMEM((CHUNK, 16), f32),
            pltpu.VMEM((CHUNK, 16), f32),
            pltpu.VMEM((CHUNK, 16), f32),
            pltpu.VMEM((CHUNK, 16), f32),
            pltpu.VMEM((CHUNK, 16), f32),
            pltpu.VMEM((CHUNK, 16), f32),
            pltpu.VMEM((2, HHALF), f32),
            pltpu.VMEM_SHARED((NPAD, HHALF), f32),
            pltpu.VMEM_SHARED((NPAD, 16), f32),
            pltpu.SemaphoreType.DMA,
            pltpu.SemaphoreType.DMA,
            pltpu.SemaphoreType.DMA,
            pltpu.SemaphoreType.DMA,
            pltpu.SemaphoreType.DMA,
            pltpu.SemaphoreType.DMA,
            pltpu.SemaphoreType.DMA,
            pltpu.SemaphoreType.DMA,
            pltpu.SemaphoreType.DMA,
            pltpu.SemaphoreType.DMA,
        ),
        compiler_params=pltpu.CompilerParams(use_tc_tiling_on_sc=False, needs_layout_passes=False),
    )
    return call(src, dst, f_all, el_all, er_all, b_all)


def _head_body(h_ref, w1_ref, b1_ref, w2_ref, b2_ref, w3_ref, b3_ref,
               o_ref, acc_ref):
    t = pl.program_id(0)
    g = t // NBLK
    i = t % NBLK

    @pl.when(t == 0)
    def _():
        acc_ref[...] = jnp.zeros_like(acc_ref)

    rows = jax.lax.broadcasted_iota(jnp.int32, (BN, 1), 0) + i * BN
    blk = jnp.where(rows < N, h_ref[...], 0.0)
    acc_ref[0:1, pl.ds(g * HHALF, HHALF)] += jnp.sum(blk, axis=0, keepdims=True)

    @pl.when(t == 2 * NBLK - 1)
    def _():
        hg = acc_ref[...] * (1.0 / N)
        o = jnp.dot(hg, w1_ref[...], preferred_element_type=jnp.float32)
        o = jnp.maximum(o + b1_ref[...], 0.0)
        o = jnp.maximum(jnp.dot(o, w2_ref[...], preferred_element_type=jnp.float32)
                        + b2_ref[...], 0.0)
        o_ref[...] = (jnp.dot(o, w3_ref[...], preferred_element_type=jnp.float32)
                      + b3_ref[...])


def _full(shape):
    return pl.BlockSpec(shape, lambda t: tuple(0 for _ in shape))


def _head(h, w1, b1, w2, b2, w3, b3):
    return pl.pallas_call(
        _head_body,
        grid=(2 * NBLK,),
        in_specs=[pl.BlockSpec((BN, HHALF), lambda t: (t, 0)),
                  _full((HEADS * HID, HID)),
                  _full((1, HID)),
                  _full((HID, HID)),
                  _full((1, HID)),
                  _full((HID, NCLS)),
                  _full((1, NCLS))],
        out_specs=pl.BlockSpec((1, NCLS), lambda t: (0, 0)),
        out_shape=jax.ShapeDtypeStruct((1, NCLS), jnp.float32),
        scratch_shapes=[pltpu.VMEM((1, HEADS * HID), jnp.float32)],
    )(h, w1, b1.reshape(1, HID), w2, b2.reshape(1, HID), w3,
      b3.reshape(1, NCLS))


def _attn_matrix(a, h0):
    sel = jnp.eye(HEADS, 16, k=-h0, dtype=a.dtype)
    return (sel[:, None, :] * a[:, :, None]).reshape(HEADS * HID, 16)


def _logit_weights(wc, al, ar):
    wal = jnp.stack([wc @ _attn_matrix(al, 0), wc @ _attn_matrix(al, 4)])
    war = jnp.stack([wc @ _attn_matrix(ar, 0), wc @ _attn_matrix(ar, 4)])
    return wal, war


def kernel(x, edge_index, Wc1, al1, ar1, bc1, Wc2, al2, ar2, bc2,
           W1, b1, W2, b2, W3, b3):
    src = edge_index[0]
    dst = edge_index[1]
    x_pad = jnp.pad(x, ((0, NPAD - N), (0, 0)))

    wal1, war1 = _logit_weights(Wc1, al1, ar1)
    f1, el1, er1 = _proj1(x_pad, Wc1, wal1, war1)
    h1 = _sc_edge(src, dst, f1, el1, er1, bc1.reshape(2, HHALF))

    wal2, war2 = _logit_weights(Wc2, al2, ar2)
    f2, el2, er2 = _proj2(h1, Wc2, wal2, war2)
    h2 = _sc_edge(src, dst, f2, el2, er2, bc2.reshape(2, HHALF))

    return _head(h2, W1, b1, W2, b2, W3, b3)


# speedup vs baseline: 88.7463x; 1.7270x over previous
import jax
import jax.numpy as jnp
from jax import lax
from jax.experimental import pallas as pl
from jax.experimental.pallas import tpu as pltpu
from jax.experimental.pallas import tpu_sc as plsc

N = 10000
E = 320000
IN_DIM = 128
HID = 32
HEADS = 8
NCLS = 10

HHALF = HEADS * HID // 2
NSUB = 16
EDGES_PER_SUB = E // NSUB
CHUNK = 80
NCHUNKS = EDGES_PER_SUB // CHUNK
NPAD = 10240
ROWS_PER_SUB = NPAD // NSUB
BN = 1280
NBLK = NPAD // BN


def _proj1_body(x_ref, w_ref, wal_ref, war_ref, f_ref, el_ref, er_ref):
    x = x_ref[...]
    f_ref[...] = jnp.dot(x, w_ref[...], preferred_element_type=jnp.float32)
    el_ref[...] = jnp.dot(x, wal_ref[0], preferred_element_type=jnp.float32)
    er_ref[...] = jnp.dot(x, war_ref[0], preferred_element_type=jnp.float32)


def _proj1(x_pad, w, wal, war):
    f32 = jnp.float32
    return pl.pallas_call(
        _proj1_body,
        grid=(2 * NBLK,),
        in_specs=[pl.BlockSpec((BN, IN_DIM), lambda t: (t % NBLK, 0)),
                  pl.BlockSpec((IN_DIM, HHALF), lambda t: (0, t // NBLK)),
                  pl.BlockSpec((1, IN_DIM, 16), lambda t: (t // NBLK, 0, 0)),
                  pl.BlockSpec((1, IN_DIM, 16), lambda t: (t // NBLK, 0, 0))],
        out_specs=[pl.BlockSpec((BN, HHALF), lambda t: (t, 0)),
                   pl.BlockSpec((BN, 16), lambda t: (t, 0)),
                   pl.BlockSpec((BN, 16), lambda t: (t, 0))],
        out_shape=[jax.ShapeDtypeStruct((2 * NPAD, HHALF), f32),
                   jax.ShapeDtypeStruct((2 * NPAD, 16), f32),
                   jax.ShapeDtypeStruct((2 * NPAD, 16), f32)],
    )(x_pad, w, wal, war)


def _proj2_body(hlo_ref, hhi_ref, w_ref, wal_ref, war_ref,
                f_ref, el_ref, er_ref):
    hlo = hlo_ref[...]
    hhi = hhi_ref[...]
    f_ref[...] = (
        jnp.dot(hlo, w_ref[:HHALF], preferred_element_type=jnp.float32)
        + jnp.dot(hhi, w_ref[HHALF:], preferred_element_type=jnp.float32))
    el_ref[...] = (
        jnp.dot(hlo, wal_ref[0, :HHALF], preferred_element_type=jnp.float32)
        + jnp.dot(hhi, wal_ref[0, HHALF:], preferred_element_type=jnp.float32))
    er_ref[...] = (
        jnp.dot(hlo, war_ref[0, :HHALF], preferred_element_type=jnp.float32)
        + jnp.dot(hhi, war_ref[0, HHALF:], preferred_element_type=jnp.float32))


def _proj2(h, w, wal, war):
    f32 = jnp.float32
    return pl.pallas_call(
        _proj2_body,
        grid=(2 * NBLK,),
        in_specs=[pl.BlockSpec((BN, HHALF), lambda t: (t % NBLK, 0)),
                  pl.BlockSpec((BN, HHALF), lambda t: (NBLK + t % NBLK, 0)),
                  pl.BlockSpec((HEADS * HID, HHALF), lambda t: (0, t // NBLK)),
                  pl.BlockSpec((1, HEADS * HID, 16), lambda t: (t // NBLK, 0, 0)),
                  pl.BlockSpec((1, HEADS * HID, 16), lambda t: (t // NBLK, 0, 0))],
        out_specs=[pl.BlockSpec((BN, HHALF), lambda t: (t, 0)),
                   pl.BlockSpec((BN, 16), lambda t: (t, 0)),
                   pl.BlockSpec((BN, 16), lambda t: (t, 0))],
        out_shape=[jax.ShapeDtypeStruct((2 * NPAD, HHALF), f32),
                   jax.ShapeDtypeStruct((2 * NPAD, 16), f32),
                   jax.ShapeDtypeStruct((2 * NPAD, 16), f32)],
    )(h, h, w, wal, war)


def _sc_edge_body(src_hbm, dst_hbm, f_hbm, el_hbm, er_hbm, b_hbm,
                  out_hbm,
                  sidx0, sidx1, didx0, didx1, sadj0, sadj1, dadj0, dadj1,
                  rawd0, rawd1, fb0, fb1, elb0, elb1, erb0, erb1, exr0, exr1,
                  bias_v, acc_feat_sh, acc_ex_sh,
                  sf0, sf1, se0, se1, si0, si1, swf0, swf1, swe0, swe1):
    c = lax.axis_index("c")
    s = lax.axis_index("s")
    iota = lax.iota(jnp.int32, 16)
    lane_d4 = iota // 4
    lane_m4 = iota % 4
    zero16 = jnp.zeros((16,), jnp.float32)
    coff = jnp.broadcast_to(c * NPAD, (16,)).astype(jnp.int32)

    SIDX = (sidx0, sidx1)
    DIDX = (didx0, didx1)
    SADJ = (sadj0, sadj1)
    DADJ = (dadj0, dadj1)
    RAWD = (rawd0, rawd1)
    FB = (fb0, fb1)
    ELB = (elb0, elb1)
    ERB = (erb0, erb1)
    EXR = (exr0, exr1)
    SF = (sf0, sf1)
    SE = (se0, se1)
    SI = (si0, si1)
    SWF = (swf0, swf1)
    SWE = (swe0, swe1)

    pltpu.sync_copy(b_hbm, bias_v)

    def _zero_bufs(i, carry):
        for j in range(HHALF // 16):
            fb0[i, pl.ds(j * 16, 16)] = zero16
        exr0[i, pl.ds(0, 16)] = zero16
        exr1[i, pl.ds(0, 16)] = zero16
        return carry

    lax.fori_loop(0, CHUNK, _zero_bufs, 0)

    rbase = pl.multiple_of(s * ROWS_PER_SUB, CHUNK)
    for k in range(ROWS_PER_SUB // CHUNK):
        r0 = pl.multiple_of(rbase + k * CHUNK, CHUNK)
        pltpu.sync_copy(fb0, acc_feat_sh.at[pl.ds(r0, CHUNK)])
        pltpu.sync_copy(exr0, acc_ex_sh.at[pl.ds(r0, CHUNK)])
    plsc.subcore_barrier()

    ebase = s * EDGES_PER_SUB

    def chunk_off(i):
        off = jnp.minimum(ebase + i * CHUNK, E - CHUNK)
        return pl.multiple_of(off, 8)

    def issue_idx(i, j):
        pltpu.async_copy(src_hbm.at[pl.ds(chunk_off(i), CHUNK)], SIDX[j], SI[j])
        pltpu.async_copy(dst_hbm.at[pl.ds(chunk_off(i), CHUNK)], DIDX[j], SI[j])

    def wait_idx(j):
        pltpu.make_async_copy(src_hbm.at[pl.ds(0, CHUNK)], SIDX[j], SI[j]).wait()
        pltpu.make_async_copy(dst_hbm.at[pl.ds(0, CHUNK)], DIDX[j], SI[j]).wait()

    def adjust_idx(j):
        for b in range(CHUNK // 16):
            sl = pl.ds(b * 16, 16)
            sv = SIDX[j][sl]
            dv = DIDX[j][sl]
            SADJ[j][sl] = sv + coff
            DADJ[j][sl] = dv + coff
            RAWD[j][sl] = dv

    def issue_gathers(j):
        pltpu.async_copy(f_hbm.at[SADJ[j]], FB[j], SF[j])
        pltpu.async_copy(el_hbm.at[SADJ[j]], ELB[j], SE[j])
        pltpu.async_copy(er_hbm.at[DADJ[j]], ERB[j], SE[j])

    def wait_gathers(j):
        pltpu.make_async_copy(f_hbm.at[pl.ds(0, CHUNK)], FB[j], SF[j]).wait()
        pltpu.make_async_copy(el_hbm.at[pl.ds(0, CHUNK)], ELB[j], SE[j]).wait()
        pltpu.make_async_copy(er_hbm.at[pl.ds(0, CHUNK)], ERB[j], SE[j]).wait()

    def wait_scatters(j):
        pltpu.make_async_copy(
            FB[j], acc_feat_sh.at[pl.ds(0, CHUNK)], SWF[j]).wait()
        pltpu.make_async_copy(
            EXR[j], acc_ex_sh.at[pl.ds(0, CHUNK)], SWE[j]).wait()

    def _bcast_lane(vec, lane):
        idxv = jnp.full((16, 1), lane, jnp.int32)
        return lax.gather(
            vec, idxv,
            lax.GatherDimensionNumbers(
                offset_dims=(), collapsed_slice_dims=(0,),
                start_index_map=(0,)),
            slice_sizes=(1,),
            mode=lax.GatherScatterMode.PROMISE_IN_BOUNDS)

    def compute_and_scatter(j):
        def grp_body(g, carry2):
            pos = g * 4 + lane_d4
            elv = plsc.load_gather(ELB[j], [pos, lane_m4])
            erv = plsc.load_gather(ERB[j], [pos, lane_m4])
            e = elv + erv
            e = jnp.maximum(e, 0.2 * e)
            ex = jnp.exp(e)
            plsc.store_scatter(EXR[j], [pos, lane_m4], ex)
            for u in range(4):
                k2 = g * 4 + u
                for h in range(4):
                    m = _bcast_lane(ex, u * 4 + h)
                    for j2 in range(2):
                        col = h * 32 + j2 * 16
                        FB[j][k2, pl.ds(col, 16)] = (
                            FB[j][k2, pl.ds(col, 16)] * m)
            return carry2

        lax.fori_loop(0, CHUNK // 4, grp_body, 0)
        pltpu.async_copy(FB[j], acc_feat_sh.at[RAWD[j]], SWF[j], add=True)
        pltpu.async_copy(EXR[j], acc_ex_sh.at[RAWD[j]], SWE[j], add=True)

    issue_idx(0, 0)
    issue_idx(1, 1)
    wait_idx(0)
    adjust_idx(0)
    issue_gathers(0)
    wait_idx(1)
    adjust_idx(1)
    issue_gathers(1)
    wait_gathers(0)
    compute_and_scatter(0)
    issue_idx(2, 0)

    def pair_body(p, carry):
        for j, q in ((1, 0), (0, 1)):
            i = 1 + 2 * p + (1 - j)
            wait_idx(q)
            wait_scatters(q)
            adjust_idx(q)
            issue_gathers(q)
            wait_gathers(j)
            compute_and_scatter(j)
            issue_idx(i + 2, j)
        return carry

    lax.fori_loop(0, (NCHUNKS - 2) // 2, pair_body, 0)

    wait_gathers(1)
    compute_and_scatter(1)
    wait_idx(0)
    wait_scatters(0)
    wait_scatters(1)
    plsc.subcore_barrier()

    obase = pl.multiple_of(c * NPAD + rbase, CHUNK)

    def fin_body(k, carry):
        r0 = pl.multiple_of(rbase + k * CHUNK, CHUNK)
        o0 = pl.multiple_of(obase + k * CHUNK, CHUNK)
        pltpu.sync_copy(acc_feat_sh.at[pl.ds(r0, CHUNK)], fb0)
        pltpu.sync_copy(acc_ex_sh.at[pl.ds(r0, CHUNK)], exr0)

        def row_body(r, carry2):
            rr = jnp.broadcast_to(r, (16,)).astype(jnp.int32)
            for h in range(4):
                hcol = jnp.broadcast_to(jnp.int32(h), (16,))
                d = plsc.load_gather(exr0, [rr, hcol])
                d = jnp.where(d > 0.0, d, 1.0)
                for j2 in range(2):
                    col = h * 32 + j2 * 16
                    v = fb0[r, pl.ds(col, 16)] / d + bias_v[c, pl.ds(col, 16)]
                    fb1[r, pl.ds(col, 16)] = jnp.maximum(v, 0.0)
            return carry2

        lax.fori_loop(0, CHUNK, row_body, 0)
        pltpu.sync_copy(fb1, out_hbm.at[pl.ds(o0, CHUNK)])
        return carry

    lax.fori_loop(0, ROWS_PER_SUB // CHUNK, fin_body, 0)


def _sc_edge(src, dst, f_all, el_all, er_all, b_all):
    f32 = jnp.float32
    call = pl.kernel(
        _sc_edge_body,
        out_type=jax.ShapeDtypeStruct((2 * NPAD, HHALF), f32),
        mesh=plsc.VectorSubcoreMesh(core_axis_name="c", subcore_axis_name="s"),
        scratch_types=(
            pltpu.VMEM((CHUNK,), jnp.int32),
            pltpu.VMEM((CHUNK,), jnp.int32),
            pltpu.VMEM((CHUNK,), jnp.int32),
            pltpu.VMEM((CHUNK,), jnp.int32),
            pltpu.VMEM((CHUNK,), jnp.int32),
            pltpu.VMEM((CHUNK,), jnp.int32),
            pltpu.VMEM((CHUNK,), jnp.int32),
            pltpu.VMEM((CHUNK,), jnp.int32),
            pltpu.VMEM((CHUNK,), jnp.int32),
            pltpu.VMEM((CHUNK,), jnp.int32),
            pltpu.VMEM((CHUNK, HHALF), f32),
            pltpu.VMEM((CHUNK, HHALF), f32),
            pltpu.VMEM((CHUNK, 16), f32),
            pltpu.VMEM((CHUNK, 16), f32),
            pltpu.VMEM((CHUNK, 16), f32),
            pltpu.VMEM((CHUNK, 16), f32),
            pltpu.VMEM((CHUNK, 16), f32),
            pltpu.VMEM((CHUNK, 16), f32),
            pltpu.VMEM((2, HHALF), f32),
            pltpu.VMEM_SHARED((NPAD, HHALF), f32),
            pltpu.VMEM_SHARED((NPAD, 16), f32),
            pltpu.SemaphoreType.DMA,
            pltpu.SemaphoreType.DMA,
            pltpu.SemaphoreType.DMA,
            pltpu.SemaphoreType.DMA,
            pltpu.SemaphoreType.DMA,
            pltpu.SemaphoreType.DMA,
            pltpu.SemaphoreType.DMA,
            pltpu.SemaphoreType.DMA,
            pltpu.SemaphoreType.DMA,
            pltpu.SemaphoreType.DMA,
        ),
        compiler_params=pltpu.CompilerParams(use_tc_tiling_on_sc=False, needs_layout_passes=False),
    )
    return call(src, dst, f_all, el_all, er_all, b_all)


def _head_body(h_ref, w1_ref, b1_ref, w2_ref, b2_ref, w3_ref, b3_ref,
               o_ref, acc_ref):
    t = pl.program_id(0)
    g = t // NBLK
    i = t % NBLK

    @pl.when(t == 0)
    def _():
        acc_ref[...] = jnp.zeros_like(acc_ref)

    rows = jax.lax.broadcasted_iota(jnp.int32, (BN, 1), 0) + i * BN
    blk = jnp.where(rows < N, h_ref[...], 0.0)
    acc_ref[0:1, pl.ds(g * HHALF, HHALF)] += jnp.sum(blk, axis=0, keepdims=True)

    @pl.when(t == 2 * NBLK - 1)
    def _():
        hg = acc_ref[...] * (1.0 / N)
        o = jnp.dot(hg, w1_ref[...], preferred_element_type=jnp.float32)
        o = jnp.maximum(o + b1_ref[...], 0.0)
        o = jnp.maximum(jnp.dot(o, w2_ref[...], preferred_element_type=jnp.float32)
                        + b2_ref[...], 0.0)
        o_ref[...] = (jnp.dot(o, w3_ref[...], preferred_element_type=jnp.float32)
                      + b3_ref[...])


def _full(shape):
    return pl.BlockSpec(shape, lambda t: tuple(0 for _ in shape))


def _head(h, w1, b1, w2, b2, w3, b3):
    return pl.pallas_call(
        _head_body,
        grid=(2 * NBLK,),
        in_specs=[pl.BlockSpec((BN, HHALF), lambda t: (t, 0)),
                  _full((HEADS * HID, HID)),
                  _full((1, HID)),
                  _full((HID, HID)),
                  _full((1, HID)),
                  _full((HID, NCLS)),
                  _full((1, NCLS))],
        out_specs=pl.BlockSpec((1, NCLS), lambda t: (0, 0)),
        out_shape=jax.ShapeDtypeStruct((1, NCLS), jnp.float32),
        scratch_shapes=[pltpu.VMEM((1, HEADS * HID), jnp.float32)],
    )(h, w1, b1.reshape(1, HID), w2, b2.reshape(1, HID), w3,
      b3.reshape(1, NCLS))


def _attn_matrix(a, h0):
    sel = jnp.eye(HEADS, 16, k=-h0, dtype=a.dtype)
    return (sel[:, None, :] * a[:, :, None]).reshape(HEADS * HID, 16)


def _logit_weights(wc, al, ar):
    wal = jnp.stack([wc @ _attn_matrix(al, 0), wc @ _attn_matrix(al, 4)])
    war = jnp.stack([wc @ _attn_matrix(ar, 0), wc @ _attn_matrix(ar, 4)])
    return wal, war


def kernel(x, edge_index, Wc1, al1, ar1, bc1, Wc2, al2, ar2, bc2,
           W1, b1, W2, b2, W3, b3):
    src = edge_index[0]
    dst = edge_index[1]
    x_pad = jnp.pad(x, ((0, NPAD - N), (0, 0)))

    wal1, war1 = _logit_weights(Wc1, al1, ar1)
    f1, el1, er1 = _proj1(x_pad, Wc1, wal1, war1)
    h1 = _sc_edge(src, dst, f1, el1, er1, bc1.reshape(2, HHALF))

    wal2, war2 = _logit_weights(Wc2, al2, ar2)
    f2, el2, er2 = _proj2(h1, Wc2, wal2, war2)
    h2 = _sc_edge(src, dst, f2, el2, er2, bc2.reshape(2, HHALF))

    return _head(h2, W1, b1, W2, b2, W3, b3)
